# baseline (speedup 1.0000x reference)
import functools

import jax
import jax.numpy as jnp
from jax import lax
from jax.experimental import pallas as pl
from jax.experimental.pallas import tpu as pltpu

_BLOCK = 4096


def _mlp_body(nrows, table_ref, w1_ref, b1_ref, w2_ref, b2_ref, out_ref):
    x = table_ref[...]
    h = jnp.dot(x, w1_ref[...], preferred_element_type=jnp.float32) + b1_ref[...]
    h = h * (1.0 / (1.0 + jnp.exp(-h)))
    y = jnp.dot(h, w2_ref[...], preferred_element_type=jnp.float32) + b2_ref[...]
    y = y * (1.0 / (1.0 + jnp.exp(-y)))
    row = lax.broadcasted_iota(jnp.int32, y.shape, 0)
    out_ref[...] = jnp.where(row < nrows, y, 0.0).astype(jnp.bfloat16)


def _gather_body(t_ref, ytab_ref, out_ref):
    idx = t_ref[0, 0, :]
    vpad = ytab_ref.shape[0]
    block = out_ref.shape[0]
    iota = lax.broadcasted_iota(jnp.int16, (block, vpad), 1)
    cond = idx.astype(jnp.int16)[:, None] == iota
    onehot = jnp.where(cond, jnp.bfloat16(1), jnp.bfloat16(0))
    out_ref[...] = jnp.dot(onehot, ytab_ref[...],
                           preferred_element_type=jnp.float32)


def kernel(t, table, W1, b1, W2, b2):
    V, D = table.shape
    P = W2.shape[1]
    B = t.shape[0]
    vpad = (V + 127) // 128 * 128
    nb = B // _BLOCK
    t3 = t.reshape(nb, 1, _BLOCK)
    ytab = pl.pallas_call(
        functools.partial(_mlp_body, V),
        grid=(1,),
        in_specs=[
            pl.BlockSpec((vpad, D), lambda b: (0, 0)),
            pl.BlockSpec((D, P), lambda b: (0, 0)),
            pl.BlockSpec((1, P), lambda b: (0, 0)),
            pl.BlockSpec((P, P), lambda b: (0, 0)),
            pl.BlockSpec((1, P), lambda b: (0, 0)),
        ],
        out_specs=pl.BlockSpec((vpad, P), lambda b: (0, 0)),
        out_shape=jax.ShapeDtypeStruct((vpad, P), jnp.bfloat16),
    )(table, W1, b1.reshape(1, -1), W2, b2.reshape(1, -1))
    return pl.pallas_call(
        _gather_body,
        grid=(nb,),
        in_specs=[
            pl.BlockSpec((1, 1, _BLOCK), lambda b: (b, 0, 0)),
            pl.BlockSpec((vpad, P), lambda b: (0, 0)),
        ],
        out_specs=pl.BlockSpec((_BLOCK, P), lambda b: (b, 0)),
        out_shape=jax.ShapeDtypeStruct((B, P), jnp.float32),
    )(t3, ytab)


# speedup vs baseline: 1.1158x; 1.1158x over previous
import functools

import jax
import jax.numpy as jnp
from jax import lax
from jax.experimental import pallas as pl
from jax.experimental.pallas import tpu as pltpu

_BLOCK = 4096


def _fused_body(nrows, t_ref, table_ref, w1_ref, b1_ref, w2_ref, b2_ref,
                out_ref, ytab_ref):
    @pl.when(pl.program_id(0) == 0)
    def _():
        x = table_ref[...]
        h = jnp.dot(x, w1_ref[...], preferred_element_type=jnp.float32) + b1_ref[...]
        h = h * (1.0 / (1.0 + jnp.exp(-h)))
        y = jnp.dot(h, w2_ref[...], preferred_element_type=jnp.float32) + b2_ref[...]
        y = y * (1.0 / (1.0 + jnp.exp(-y)))
        row = lax.broadcasted_iota(jnp.int32, y.shape, 0)
        ytab_ref[...] = jnp.where(row < nrows, y, 0.0).astype(jnp.bfloat16)

    idx = t_ref[0, 0, :]
    vpad = ytab_ref.shape[0]
    block = out_ref.shape[0]
    iota = lax.broadcasted_iota(jnp.int16, (block, vpad), 1)
    cond = idx.astype(jnp.int16)[:, None] == iota
    onehot = jnp.where(cond, jnp.bfloat16(1), jnp.bfloat16(0))
    out_ref[...] = jnp.dot(onehot, ytab_ref[...],
                           preferred_element_type=jnp.float32)


def kernel(t, table, W1, b1, W2, b2):
    V, D = table.shape
    P = W2.shape[1]
    B = t.shape[0]
    vpad = (V + 127) // 128 * 128
    nb = B // _BLOCK
    t3 = t.reshape(nb, 1, _BLOCK)
    return pl.pallas_call(
        functools.partial(_fused_body, V),
        grid=(nb,),
        in_specs=[
            pl.BlockSpec((1, 1, _BLOCK), lambda b: (b, 0, 0)),
            pl.BlockSpec((vpad, D), lambda b: (0, 0)),
            pl.BlockSpec((D, P), lambda b: (0, 0)),
            pl.BlockSpec((1, P), lambda b: (0, 0)),
            pl.BlockSpec((P, P), lambda b: (0, 0)),
            pl.BlockSpec((1, P), lambda b: (0, 0)),
        ],
        out_specs=pl.BlockSpec((_BLOCK, P), lambda b: (b, 0)),
        out_shape=jax.ShapeDtypeStruct((B, P), jnp.float32),
        scratch_shapes=[pltpu.VMEM((vpad, P), jnp.bfloat16)],
    )(t3, table, W1, b1.reshape(1, -1), W2, b2.reshape(1, -1))


# speedup vs baseline: 1.1160x; 1.0002x over previous
import functools

import jax
import jax.numpy as jnp
from jax import lax
from jax.experimental import pallas as pl
from jax.experimental.pallas import tpu as pltpu

_BLOCK = 4096


def _fused_body(nrows, t_ref, table_ref, w1_ref, b1_ref, w2_ref, b2_ref,
                out_ref, ytab_ref):
    @pl.when(pl.program_id(0) == 0)
    def _():
        x = table_ref[...]
        h = jnp.dot(x, w1_ref[...], preferred_element_type=jnp.float32) + b1_ref[...]
        h = h * (1.0 / (1.0 + jnp.exp(-h)))
        y = jnp.dot(h, w2_ref[...], preferred_element_type=jnp.float32) + b2_ref[...]
        y = y * (1.0 / (1.0 + jnp.exp(-y)))
        row = lax.broadcasted_iota(jnp.int32, y.shape, 0)
        ytab_ref[...] = jnp.where(row < nrows, y, 0.0).astype(jnp.bfloat16)

    idx = t_ref[0, 0, :]
    vpad = ytab_ref.shape[0]
    block = out_ref.shape[0]
    iota = lax.broadcasted_iota(jnp.int16, (block, vpad), 1)
    cond = idx.astype(jnp.int16)[:, None] == iota
    onehot = jnp.where(cond, jnp.bfloat16(1), jnp.bfloat16(0))
    out_ref[...] = jnp.dot(onehot, ytab_ref[...],
                           preferred_element_type=jnp.float32)


def kernel(t, table, W1, b1, W2, b2):
    V, D = table.shape
    P = W2.shape[1]
    B = t.shape[0]
    vpad = (V + 127) // 128 * 128
    nb = B // _BLOCK
    t3 = t.reshape(nb, 1, _BLOCK)
    return pl.pallas_call(
        functools.partial(_fused_body, V),
        grid=(nb,),
        in_specs=[
            pl.BlockSpec((1, 1, _BLOCK), lambda b: (b, 0, 0)),
            pl.BlockSpec((vpad, D), lambda b: (0, 0)),
            pl.BlockSpec((D, P), lambda b: (0, 0)),
            pl.BlockSpec((1, P), lambda b: (0, 0)),
            pl.BlockSpec((P, P), lambda b: (0, 0)),
            pl.BlockSpec((1, P), lambda b: (0, 0)),
        ],
        out_specs=pl.BlockSpec((_BLOCK, P), lambda b: (b, 0)),
        out_shape=jax.ShapeDtypeStruct((B, P), jnp.float32),
        scratch_shapes=[pltpu.VMEM((vpad, P), jnp.bfloat16)],
        compiler_params=pltpu.CompilerParams(
            dimension_semantics=("arbitrary",),
            vmem_limit_bytes=100 * 1024 * 1024,
        ),
    )(t3, table, W1, b1.reshape(1, -1), W2, b2.reshape(1, -1))
